# Initial kernel scaffold; baseline (speedup 1.0000x reference)
import jax
import jax.numpy as jnp
from jax.experimental import pallas as pl


def kernel(seqs, seqL):
    raise NotImplementedError("write your pallas kernel here")


# speedup vs baseline: 1.1085x; 1.1085x over previous
import jax
import jax.numpy as jnp
from jax import lax
from jax.experimental import pallas as pl
from jax.experimental.pallas import tpu as pltpu
from jax.experimental.pallas import tpu_sc as plsc

N, C, S = 1, 1024, 16384
B = 8
L = 16
NC, NS = 2, 16
NW = NC * NS
RPW = C // NW
NEG = float("-inf")


def _segmax_body(seqs_hbm, seqL_hbm, out_hbm, len_v, row_v, res_v, outT_v, sem):
    wid = lax.axis_index("c") * NS + lax.axis_index("s")
    row0 = wid * RPW

    pltpu.sync_copy(seqL_hbm, len_v.at[pl.ds(0, B)])

    lane = lax.broadcasted_iota(jnp.int32, (L,), 0)
    neg = jnp.full((L,), NEG, jnp.float32)

    lens = len_v[...]
    starts, ends = [], []
    acc = jnp.int32(0)
    for b in range(B):
        starts.append(acc)
        acc = acc + lens[b]
        ends.append(acc)

    def lane_max(a):
        for sh in (8, 4, 2, 1):
            a = jnp.maximum(a, a.at[lane ^ sh].get(mode="promise_in_bounds"))
        return a

    def do_row(i, _):
        pltpu.sync_copy(seqs_hbm.at[row0 + i], row_v)
        res = neg
        for b in range(B):
            st, en = starts[b], ends[b]
            j0 = lax.shift_right_logical(st, 4)
            j1 = lax.shift_right_logical(en - 1, 4)
            idx0 = j0 * L + lane
            a = jnp.where((idx0 >= st) & (idx0 < en), row_v[pl.ds(j0 * L, L)], neg)
            idx1 = j1 * L + lane
            a = jnp.maximum(
                a, jnp.where((idx1 >= st) & (idx1 < en), row_v[pl.ds(j1 * L, L)], neg)
            )

            def inner(j, acc):
                return jnp.maximum(acc, row_v[pl.ds(j * L, L)])

            a = lax.fori_loop(j0 + 1, jnp.maximum(j0 + 1, j1), inner, a)
            res = jnp.where(lane == b, lane_max(a), res)
        res_v[pl.ds(i * L, L)] = res
        return 0

    lax.fori_loop(0, RPW, do_row, 0)

    for b in range(B):
        for h in range(0, RPW, L):
            idx = (h + lane) * L + b
            outT_v[b, pl.ds(h, L)] = plsc.load_gather(res_v, [idx])
    for b in range(B):
        pltpu.sync_copy(outT_v.at[b], out_hbm.at[b, pl.ds(row0, RPW)])


@jax.jit
def _segmax(seqs2, seqL1):
    mesh = plsc.VectorSubcoreMesh(
        core_axis_name="c", subcore_axis_name="s", num_cores=NC, num_subcores=NS
    )
    return pl.kernel(
        _segmax_body,
        out_type=jax.ShapeDtypeStruct((B, C), jnp.float32),
        mesh=mesh,
        compiler_params=pltpu.CompilerParams(needs_layout_passes=False),
        scratch_types=[
            pltpu.VMEM((L,), jnp.int32),
            pltpu.VMEM((S,), jnp.float32),
            pltpu.VMEM((RPW * L,), jnp.float32),
            pltpu.VMEM((B, RPW), jnp.float32),
            pltpu.SemaphoreType.DMA,
        ],
    )(seqs2, seqL1)


def kernel(seqs, seqL):
    seqs2 = seqs.reshape(C, S)
    seqL1 = seqL.reshape(B).astype(jnp.int32)
    return _segmax(seqs2, seqL1)


# speedup vs baseline: 1.7613x; 1.5889x over previous
import jax
import jax.numpy as jnp
from jax import lax
from jax.experimental import pallas as pl
from jax.experimental.pallas import tpu as pltpu
from jax.experimental.pallas import tpu_sc as plsc

N, C, S = 1, 1024, 16384
B = 8
L = 16
NC, NS = 2, 16
NW = NC * NS
RPW = C // NW
NEG = float("-inf")


def _segmax_body(seqs_hbm, seqL_hbm, out_hbm, len_v, row_v, res_v, outT_v, sem):
    wid = lax.axis_index("c") * NS + lax.axis_index("s")
    row0 = wid * RPW

    pltpu.sync_copy(seqL_hbm, len_v.at[pl.ds(0, B)])

    lane = lax.broadcasted_iota(jnp.int32, (L,), 0)
    neg = jnp.full((L,), NEG, jnp.float32)

    lens = len_v[...]
    starts, ends = [], []
    acc = jnp.int32(0)
    for b in range(B):
        starts.append(acc)
        acc = acc + lens[b]
        ends.append(acc)

    def lane_max(a):
        for sh in (8, 4, 2, 1):
            a = jnp.maximum(a, a.at[lane ^ sh].get(mode="promise_in_bounds"))
        return a

    def do_row(i, _):
        pltpu.sync_copy(seqs_hbm.at[row0 + i], row_v)
        res = neg
        for b in range(B):
            st, en = starts[b], ends[b]
            j0 = lax.shift_right_logical(st, 4)
            j1 = lax.shift_right_logical(en - 1, 4)
            idx0 = j0 * L + lane
            a = jnp.where((idx0 >= st) & (idx0 < en), row_v[pl.ds(j0 * L, L)], neg)
            idx1 = j1 * L + lane
            a = jnp.maximum(
                a, jnp.where((idx1 >= st) & (idx1 < en), row_v[pl.ds(j1 * L, L)], neg)
            )

            U = 8
            lo = j0 + 1
            hi = jnp.maximum(lo, j1)
            n_blk = (hi - lo) // U

            def blk(k, acc):
                base = (lo + k * U) * L
                v = [row_v[pl.ds(base + u * L, L)] for u in range(U)]
                m = [jnp.maximum(v[2 * u], v[2 * u + 1]) for u in range(U // 2)]
                m = [jnp.maximum(m[2 * u], m[2 * u + 1]) for u in range(U // 4)]
                return jnp.maximum(acc, jnp.maximum(m[0], m[1]))

            a = lax.fori_loop(0, n_blk, blk, a)

            def rem(j, acc):
                return jnp.maximum(acc, row_v[pl.ds(j * L, L)])

            a = lax.fori_loop(lo + n_blk * U, hi, rem, a)
            res = jnp.where(lane == b, lane_max(a), res)
        res_v[pl.ds(i * L, L)] = res
        return 0

    lax.fori_loop(0, RPW, do_row, 0)

    for b in range(B):
        for h in range(0, RPW, L):
            idx = (h + lane) * L + b
            outT_v[b, pl.ds(h, L)] = plsc.load_gather(res_v, [idx])
    for b in range(B):
        pltpu.sync_copy(outT_v.at[b], out_hbm.at[b, pl.ds(row0, RPW)])


@jax.jit
def _segmax(seqs2, seqL1):
    mesh = plsc.VectorSubcoreMesh(
        core_axis_name="c", subcore_axis_name="s", num_cores=NC, num_subcores=NS
    )
    return pl.kernel(
        _segmax_body,
        out_type=jax.ShapeDtypeStruct((B, C), jnp.float32),
        mesh=mesh,
        compiler_params=pltpu.CompilerParams(needs_layout_passes=False),
        scratch_types=[
            pltpu.VMEM((L,), jnp.int32),
            pltpu.VMEM((S,), jnp.float32),
            pltpu.VMEM((RPW * L,), jnp.float32),
            pltpu.VMEM((B, RPW), jnp.float32),
            pltpu.SemaphoreType.DMA,
        ],
    )(seqs2, seqL1)


def kernel(seqs, seqL):
    seqs2 = seqs.reshape(C, S)
    seqL1 = seqL.reshape(B).astype(jnp.int32)
    return _segmax(seqs2, seqL1)


# speedup vs baseline: 2.6708x; 1.5164x over previous
import jax
import jax.numpy as jnp
from jax import lax
from jax.experimental import pallas as pl
from jax.experimental.pallas import tpu as pltpu
from jax.experimental.pallas import tpu_sc as plsc

N, C, S = 1, 1024, 16384
B = 8
L = 16
NC, NS = 2, 16
NW = NC * NS
RPW = C // NW
NEG = float("-inf")


def _segmax_body(seqs_hbm, seqL_hbm, out_hbm, len_v, row_v, res_v, outT_v, sem):
    wid = lax.axis_index("c") * NS + lax.axis_index("s")
    row0 = wid * RPW

    pltpu.sync_copy(seqL_hbm, len_v.at[pl.ds(0, B)])

    lane = lax.broadcasted_iota(jnp.int32, (L,), 0)
    neg = jnp.full((L,), NEG, jnp.float32)

    lens = len_v[...]
    starts, ends = [], []
    acc = jnp.int32(0)
    for b in range(B):
        starts.append(acc)
        acc = acc + lens[b]
        ends.append(acc)

    def lane_max(a):
        for sh in (8, 4, 2, 1):
            a = jnp.maximum(a, a.at[lane ^ sh].get(mode="promise_in_bounds"))
        return a

    def start_row(i):
        slot = lax.rem(i, 2)
        pltpu.make_async_copy(
            seqs_hbm.at[row0 + i], row_v.at[pl.ds(slot * S, S)], sem.at[slot]
        ).start()

    start_row(jnp.int32(0))

    def do_row(i, _):
        @pl.when(i + 1 < RPW)
        def _():
            start_row(i + 1)

        slot = lax.rem(i, 2)
        row = row_v.at[pl.ds(slot * S, S)]
        pltpu.make_async_copy(seqs_hbm.at[row0 + i], row, sem.at[slot]).wait()
        res = neg
        for b in range(B):
            st, en = starts[b], ends[b]
            j0 = lax.shift_right_logical(st, 4)
            j1 = lax.shift_right_logical(en - 1, 4)
            idx0 = j0 * L + lane
            a = jnp.where((idx0 >= st) & (idx0 < en), row[pl.ds(j0 * L, L)], neg)
            idx1 = j1 * L + lane
            a = jnp.maximum(
                a, jnp.where((idx1 >= st) & (idx1 < en), row[pl.ds(j1 * L, L)], neg)
            )

            U = 8
            lo = j0 + 1
            hi = jnp.maximum(lo, j1)
            n_blk = (hi - lo) // U

            def blk(k, acc):
                base = (lo + k * U) * L
                v = [row[pl.ds(base + u * L, L)] for u in range(U)]
                m = [jnp.maximum(v[2 * u], v[2 * u + 1]) for u in range(U // 2)]
                m = [jnp.maximum(m[2 * u], m[2 * u + 1]) for u in range(U // 4)]
                return jnp.maximum(acc, jnp.maximum(m[0], m[1]))

            a = lax.fori_loop(0, n_blk, blk, a)

            def rem(j, acc):
                return jnp.maximum(acc, row[pl.ds(j * L, L)])

            a = lax.fori_loop(lo + n_blk * U, hi, rem, a)
            res = jnp.where(lane == b, lane_max(a), res)
        res_v[pl.ds(i * L, L)] = res
        return 0

    lax.fori_loop(0, RPW, do_row, 0)

    for b in range(B):
        for h in range(0, RPW, L):
            idx = (h + lane) * L + b
            outT_v[b, pl.ds(h, L)] = plsc.load_gather(res_v, [idx])
    for b in range(B):
        pltpu.sync_copy(outT_v.at[b], out_hbm.at[b, pl.ds(row0, RPW)])


@jax.jit
def _segmax(seqs2, seqL1):
    mesh = plsc.VectorSubcoreMesh(
        core_axis_name="c", subcore_axis_name="s", num_cores=NC, num_subcores=NS
    )
    return pl.kernel(
        _segmax_body,
        out_type=jax.ShapeDtypeStruct((B, C), jnp.float32),
        mesh=mesh,
        compiler_params=pltpu.CompilerParams(needs_layout_passes=False),
        scratch_types=[
            pltpu.VMEM((L,), jnp.int32),
            pltpu.VMEM((2 * S,), jnp.float32),
            pltpu.VMEM((RPW * L,), jnp.float32),
            pltpu.VMEM((B, RPW), jnp.float32),
            pltpu.SemaphoreType.DMA((2,)),
        ],
    )(seqs2, seqL1)


def kernel(seqs, seqL):
    seqs2 = seqs.reshape(C, S)
    seqL1 = seqL.reshape(B).astype(jnp.int32)
    return _segmax(seqs2, seqL1)


# speedup vs baseline: 2.9424x; 1.1017x over previous
import jax
import jax.numpy as jnp
from jax import lax
from jax.experimental import pallas as pl
from jax.experimental.pallas import tpu as pltpu
from jax.experimental.pallas import tpu_sc as plsc

N, C, S = 1, 1024, 16384
B = 8
L = 16
NC, NS = 2, 16
NW = NC * NS
RPW = C // NW
NEG = float("-inf")


def _segmax_body(seqs_hbm, seqL_hbm, out_hbm, len_v, row_v, res_v, outT_v, sem):
    wid = lax.axis_index("c") * NS + lax.axis_index("s")
    row0 = wid * RPW

    pltpu.sync_copy(seqL_hbm, len_v.at[pl.ds(0, B)])

    lane = lax.broadcasted_iota(jnp.int32, (L,), 0)
    neg = jnp.full((L,), NEG, jnp.float32)

    lens = len_v[...]
    starts, ends = [], []
    acc = jnp.int32(0)
    for b in range(B):
        starts.append(acc)
        acc = acc + lens[b]
        ends.append(acc)

    def lane_max(a):
        for sh in (8, 4, 2, 1):
            a = jnp.maximum(a, a.at[lane ^ sh].get(mode="promise_in_bounds"))
        return a

    total = ends[B - 1]
    CH = 1024
    n_ch = (total + (CH - 1)) // CH

    def chunk_copy(i, slot, k):
        return pltpu.make_async_copy(
            seqs_hbm.at[row0 + i, pl.ds(k * CH, CH)],
            row_v.at[pl.ds(slot * S + k * CH, CH)],
            sem.at[slot],
        )

    def start_row(i):
        slot = lax.rem(i, 2)

        def go(k, _):
            chunk_copy(i, slot, k).start()
            return 0

        lax.fori_loop(0, n_ch, go, 0)

    start_row(jnp.int32(0))

    def do_row(i, _):
        @pl.when(i + 1 < RPW)
        def _():
            start_row(i + 1)

        slot = lax.rem(i, 2)
        row = row_v.at[pl.ds(slot * S, S)]

        def drain(k, _):
            chunk_copy(i, slot, k).wait()
            return 0

        lax.fori_loop(0, n_ch, drain, 0)
        res = neg
        for b in range(B):
            st, en = starts[b], ends[b]
            j0 = lax.shift_right_logical(st, 4)
            j1 = lax.shift_right_logical(en - 1, 4)
            idx0 = j0 * L + lane
            a = jnp.where((idx0 >= st) & (idx0 < en), row[pl.ds(j0 * L, L)], neg)
            idx1 = j1 * L + lane
            a = jnp.maximum(
                a, jnp.where((idx1 >= st) & (idx1 < en), row[pl.ds(j1 * L, L)], neg)
            )

            U = 8
            lo = j0 + 1
            hi = jnp.maximum(lo, j1)
            n_blk = (hi - lo) // U

            def blk(k, acc):
                base = (lo + k * U) * L
                v = [row[pl.ds(base + u * L, L)] for u in range(U)]
                m = [jnp.maximum(v[2 * u], v[2 * u + 1]) for u in range(U // 2)]
                m = [jnp.maximum(m[2 * u], m[2 * u + 1]) for u in range(U // 4)]
                return jnp.maximum(acc, jnp.maximum(m[0], m[1]))

            a = lax.fori_loop(0, n_blk, blk, a)

            def rem(j, acc):
                return jnp.maximum(acc, row[pl.ds(j * L, L)])

            a = lax.fori_loop(lo + n_blk * U, hi, rem, a)
            res = jnp.where(lane == b, lane_max(a), res)
        res_v[pl.ds(i * L, L)] = res
        return 0

    lax.fori_loop(0, RPW, do_row, 0)

    for b in range(B):
        for h in range(0, RPW, L):
            idx = (h + lane) * L + b
            outT_v[b, pl.ds(h, L)] = plsc.load_gather(res_v, [idx])
    for b in range(B):
        pltpu.sync_copy(outT_v.at[b], out_hbm.at[b, pl.ds(row0, RPW)])


@jax.jit
def _segmax(seqs2, seqL1):
    mesh = plsc.VectorSubcoreMesh(
        core_axis_name="c", subcore_axis_name="s", num_cores=NC, num_subcores=NS
    )
    return pl.kernel(
        _segmax_body,
        out_type=jax.ShapeDtypeStruct((B, C), jnp.float32),
        mesh=mesh,
        compiler_params=pltpu.CompilerParams(needs_layout_passes=False),
        scratch_types=[
            pltpu.VMEM((L,), jnp.int32),
            pltpu.VMEM((2 * S,), jnp.float32),
            pltpu.VMEM((RPW * L,), jnp.float32),
            pltpu.VMEM((B, RPW), jnp.float32),
            pltpu.SemaphoreType.DMA((2,)),
        ],
    )(seqs2, seqL1)


def kernel(seqs, seqL):
    seqs2 = seqs.reshape(C, S)
    seqL1 = seqL.reshape(B).astype(jnp.int32)
    return _segmax(seqs2, seqL1)


# speedup vs baseline: 2.9966x; 1.0184x over previous
import jax
import jax.numpy as jnp
from jax import lax
from jax.experimental import pallas as pl
from jax.experimental.pallas import tpu as pltpu
from jax.experimental.pallas import tpu_sc as plsc

N, C, S = 1, 1024, 16384
B = 8
L = 16
NC, NS = 2, 16
NW = NC * NS
RPW = C // NW
NEG = float("-inf")
SP = S + 8 * L


def _segmax_body(seqs_hbm, seqL_hbm, out_hbm, len_v, row_v, res_v, outT_v, sem):
    wid = lax.axis_index("c") * NS + lax.axis_index("s")
    row0 = wid * RPW

    pltpu.sync_copy(seqL_hbm, len_v.at[pl.ds(0, B)])

    lane = lax.broadcasted_iota(jnp.int32, (L,), 0)
    neg = jnp.full((L,), NEG, jnp.float32)

    lens = len_v[...]
    starts, ends = [], []
    acc = jnp.int32(0)
    for b in range(B):
        starts.append(acc)
        acc = acc + lens[b]
        ends.append(acc)

    def lane_max(a):
        for sh in (8, 4, 2, 1):
            a = jnp.maximum(a, a.at[lane ^ sh].get(mode="promise_in_bounds"))
        return a

    total = ends[B - 1]
    CH = 1024
    n_ch = (total + (CH - 1)) // CH

    def chunk_copy(i, slot, k):
        return pltpu.make_async_copy(
            seqs_hbm.at[row0 + i, pl.ds(k * CH, CH)],
            row_v.at[pl.ds(slot * SP + k * CH, CH)],
            sem.at[slot],
        )

    def start_row(i):
        slot = lax.rem(i, 2)

        def go(k, _):
            chunk_copy(i, slot, k).start()
            return 0

        lax.fori_loop(0, n_ch, go, 0)

    start_row(jnp.int32(0))

    def do_row(i, _):
        @pl.when(i + 1 < RPW)
        def _():
            start_row(i + 1)

        slot = lax.rem(i, 2)
        row = row_v.at[pl.ds(slot * SP, SP)]

        def drain(k, _):
            chunk_copy(i, slot, k).wait()
            return 0

        lax.fori_loop(0, n_ch, drain, 0)
        res = neg
        for b in range(B):
            st, en = starts[b], ends[b]
            j0 = lax.shift_right_logical(st, 4)
            j1 = lax.shift_right_logical(en - 1, 4)
            idx0 = j0 * L + lane
            a = jnp.where((idx0 >= st) & (idx0 < en), row[pl.ds(j0 * L, L)], neg)
            idx1 = j1 * L + lane
            a = jnp.maximum(
                a, jnp.where((idx1 >= st) & (idx1 < en), row[pl.ds(j1 * L, L)], neg)
            )

            U = 8
            lo = j0 + 1
            n = j1 - lo

            def tree(v):
                m = [jnp.maximum(v[2 * u], v[2 * u + 1]) for u in range(U // 2)]
                m = [jnp.maximum(m[2 * u], m[2 * u + 1]) for u in range(U // 4)]
                return jnp.maximum(m[0], m[1])

            def big(acc):
                def blk(k, acc):
                    base = (lo + k * U) * L
                    return jnp.maximum(
                        acc, tree([row[pl.ds(base + u * L, L)] for u in range(U)])
                    )

                acc = lax.fori_loop(0, n // U, blk, acc)
                base = (j1 - U) * L
                return jnp.maximum(
                    acc, tree([row[pl.ds(base + u * L, L)] for u in range(U)])
                )

            def small(acc):
                vals = [
                    jnp.where(lo + u < j1, row[pl.ds((lo + u) * L, L)], neg)
                    for u in range(U)
                ]
                return jnp.maximum(acc, tree(vals))

            a = lax.cond(n >= U, big, small, a)
            res = jnp.where(lane == b, lane_max(a), res)
        res_v[pl.ds(i * L, L)] = res
        return 0

    lax.fori_loop(0, RPW, do_row, 0)

    for b in range(B):
        for h in range(0, RPW, L):
            idx = (h + lane) * L + b
            outT_v[b, pl.ds(h, L)] = plsc.load_gather(res_v, [idx])
    for b in range(B):
        pltpu.sync_copy(outT_v.at[b], out_hbm.at[b, pl.ds(row0, RPW)])


@jax.jit
def _segmax(seqs2, seqL1):
    mesh = plsc.VectorSubcoreMesh(
        core_axis_name="c", subcore_axis_name="s", num_cores=NC, num_subcores=NS
    )
    return pl.kernel(
        _segmax_body,
        out_type=jax.ShapeDtypeStruct((B, C), jnp.float32),
        mesh=mesh,
        compiler_params=pltpu.CompilerParams(needs_layout_passes=False),
        scratch_types=[
            pltpu.VMEM((L,), jnp.int32),
            pltpu.VMEM((2 * SP,), jnp.float32),
            pltpu.VMEM((RPW * L,), jnp.float32),
            pltpu.VMEM((B, RPW), jnp.float32),
            pltpu.SemaphoreType.DMA((2,)),
        ],
    )(seqs2, seqL1)


def kernel(seqs, seqL):
    seqs2 = seqs.reshape(C, S)
    seqL1 = seqL.reshape(B).astype(jnp.int32)
    return _segmax(seqs2, seqL1)


# speedup vs baseline: 3.7961x; 1.2668x over previous
import functools

import jax
import jax.numpy as jnp
from jax import lax
from jax.experimental import pallas as pl
from jax.experimental.pallas import tpu as pltpu
from jax.experimental.pallas import tpu_sc as plsc

N, C, S = 1, 1024, 16384
B = 8
L = 16
NC, NS = 2, 16
NW = NC * NS

C_SC = 384
C_TC = C - C_SC
RPW = C_SC // NW

NEG = float("-inf")
SP = S + 8 * L


def _sc_body(seqs_hbm, seqL_hbm, out_hbm, len_v, row_v, res_v, outT_v, sem):
    wid = lax.axis_index("c") * NS + lax.axis_index("s")
    row0 = wid * RPW

    pltpu.sync_copy(seqL_hbm, len_v.at[pl.ds(0, B)])

    lane = lax.broadcasted_iota(jnp.int32, (L,), 0)
    neg = jnp.full((L,), NEG, jnp.float32)

    lens = len_v[...]
    starts, ends = [], []
    acc = jnp.int32(0)
    for b in range(B):
        starts.append(acc)
        acc = acc + lens[b]
        ends.append(acc)

    def lane_max(a):
        for sh in (8, 4, 2, 1):
            a = jnp.maximum(a, a.at[lane ^ sh].get(mode="promise_in_bounds"))
        return a

    total = ends[B - 1]
    CH = 1024
    n_ch = (total + (CH - 1)) // CH

    def chunk_copy(i, slot, k):
        return pltpu.make_async_copy(
            seqs_hbm.at[row0 + i, pl.ds(k * CH, CH)],
            row_v.at[pl.ds(slot * SP + k * CH, CH)],
            sem.at[slot],
        )

    def start_row(i):
        slot = lax.rem(i, 2)

        def go(k, _):
            chunk_copy(i, slot, k).start()
            return 0

        lax.fori_loop(0, n_ch, go, 0)

    start_row(jnp.int32(0))

    def do_row(i, _):
        @pl.when(i + 1 < RPW)
        def _():
            start_row(i + 1)

        slot = lax.rem(i, 2)
        row = row_v.at[pl.ds(slot * SP, SP)]

        def drain(k, _):
            chunk_copy(i, slot, k).wait()
            return 0

        lax.fori_loop(0, n_ch, drain, 0)
        res = neg
        for b in range(B):
            st, en = starts[b], ends[b]
            j0 = lax.shift_right_logical(st, 4)
            j1 = lax.shift_right_logical(en - 1, 4)
            idx0 = j0 * L + lane
            a = jnp.where((idx0 >= st) & (idx0 < en), row[pl.ds(j0 * L, L)], neg)
            idx1 = j1 * L + lane
            a = jnp.maximum(
                a, jnp.where((idx1 >= st) & (idx1 < en), row[pl.ds(j1 * L, L)], neg)
            )

            U = 8
            lo = j0 + 1
            n = j1 - lo

            def tree(v):
                m = [jnp.maximum(v[2 * u], v[2 * u + 1]) for u in range(U // 2)]
                m = [jnp.maximum(m[2 * u], m[2 * u + 1]) for u in range(U // 4)]
                return jnp.maximum(m[0], m[1])

            def big(acc):
                def blk(k, acc):
                    base = (lo + k * U) * L
                    return jnp.maximum(
                        acc, tree([row[pl.ds(base + u * L, L)] for u in range(U)])
                    )

                acc = lax.fori_loop(0, n // U, blk, acc)
                base = (j1 - U) * L
                return jnp.maximum(
                    acc, tree([row[pl.ds(base + u * L, L)] for u in range(U)])
                )

            def small(acc):
                vals = [
                    jnp.where(lo + u < j1, row[pl.ds((lo + u) * L, L)], neg)
                    for u in range(U)
                ]
                return jnp.maximum(acc, tree(vals))

            a = lax.cond(n >= U, big, small, a)
            res = jnp.where(lane == b, lane_max(a), res)
        res_v[pl.ds(i * L, L)] = res
        return 0

    lax.fori_loop(0, RPW, do_row, 0)

    for b in range(B):
        idx = lane * L + b
        outT_v[b, :] = plsc.load_gather(res_v, [idx])
    pltpu.sync_copy(outT_v, out_hbm.at[wid])


def _tc_body(seqL_ref, x_ref, o_ref):
    BR = x_ref.shape[0]
    PC = 512
    NP = S // PC

    starts, ends = [], []
    acc = jnp.int32(0)
    for b in range(B):
        starts.append(acc)
        acc = acc + seqL_ref[0, b]
        ends.append(acc)

    x = x_ref[...]
    pm = jnp.max(x.reshape(BR, NP, PC), axis=2)

    piece = lax.broadcasted_iota(jnp.int32, (1, NP), 1)
    col = lax.broadcasted_iota(jnp.int32, (1, PC), 1)
    neg = jnp.float32(NEG)

    for b in range(B):
        st, en = starts[b], ends[b]
        full = (piece * PC >= st) & ((piece + 1) * PC <= en)
        seg = jnp.max(jnp.where(full, pm, neg), axis=1)
        for p in (st // PC, (en - 1) // PC):
            cols = p * PC + col
            xm = jnp.where(
                (cols >= st) & (cols < en), x_ref[:, pl.ds(pl.multiple_of(p * PC, PC), PC)], neg
            )
            seg = jnp.maximum(seg, jnp.max(xm, axis=1))
        o_ref[b, :] = seg


@jax.jit
def _segmax(seqs2, seqL2):
    seqL1 = seqL2.reshape(B)
    mesh = plsc.VectorSubcoreMesh(
        core_axis_name="c", subcore_axis_name="s", num_cores=NC, num_subcores=NS
    )
    out_sc3 = pl.kernel(
        _sc_body,
        out_type=jax.ShapeDtypeStruct((NW, B, L), jnp.float32),
        mesh=mesh,
        compiler_params=pltpu.CompilerParams(needs_layout_passes=False),
        scratch_types=[
            pltpu.VMEM((L,), jnp.int32),
            pltpu.VMEM((2 * SP,), jnp.float32),
            pltpu.VMEM((L * L,), jnp.float32),
            pltpu.VMEM((B, L), jnp.float32),
            pltpu.SemaphoreType.DMA((2,)),
        ],
    )(seqs2, seqL1)

    BR = 128
    out_tc = pl.pallas_call(
        _tc_body,
        grid=(C_TC // BR,),
        in_specs=[
            pl.BlockSpec(memory_space=pltpu.SMEM),
            pl.BlockSpec((BR, S), lambda i: (C_SC // BR + i, 0)),
        ],
        out_specs=pl.BlockSpec((B, BR), lambda i: (0, i)),
        out_shape=jax.ShapeDtypeStruct((B, C_TC), jnp.float32),
    )(seqL2, seqs2)

    out_sc = out_sc3[:, :, :RPW].transpose(1, 0, 2).reshape(B, C_SC)
    return jnp.concatenate([out_sc, out_tc], axis=1)


def kernel(seqs, seqL):
    seqs2 = seqs.reshape(C, S)
    seqL2 = seqL.astype(jnp.int32)
    return _segmax(seqs2, seqL2)
